# baseline (speedup 1.0000x reference)
import functools
import math

import jax
import jax.numpy as jnp
from jax import lax
from jax.experimental import pallas as pl
from jax.experimental.pallas import tpu as pltpu
from jax.experimental.pallas import tpu_sc as plsc

N = 10000
DIM = 256
HEADS = 8
HEAD_DIM = DIM // HEADS
E = 160000

NC = 2
NS = 16
NW = NC * NS
EPW = E // NW
G = 16
CH2 = 64
NCH = (EPW + CH2 - 1) // CH2
NP2 = (NCH - 1) // 2
EBUF = NCH * CH2
CH = 32
HALF = N // NC
CAP = ((EPW + 2 * CH2 - 1) // (2 * CH2)) * (2 * CH2)
SEG = 2048
PK_TOTAL = NW * NC * CAP + SEG
INV_SQRT_HD = 1.0 / math.sqrt(HEAD_DIM)

_mesh = functools.partial(
    plsc.VectorSubcoreMesh,
    core_axis_name="c", subcore_axis_name="s", num_cores=NC, num_subcores=NS)

_sc_params = functools.partial(
    pltpu.CompilerParams, use_tc_tiling_on_sc=False, needs_layout_passes=False)


def _qkv_body(h_ref, w_ref, q_ref, k_ref, v_ref):
    y = jnp.dot(h_ref[...], w_ref[...], preferred_element_type=jnp.float32)
    q_ref[...] = y[:, :DIM]
    k_ref[...] = y[:, DIM:2 * DIM]
    v_ref[...] = y[:, 2 * DIM:]


def _qkv(h_one, W_qkv):
    BR = 2000
    return pl.pallas_call(
        _qkv_body,
        grid=(N // BR,),
        in_specs=[pl.BlockSpec((BR, DIM), lambda i: (i, 0)),
                  pl.BlockSpec((DIM, 3 * DIM), lambda i: (0, 0))],
        out_specs=[pl.BlockSpec((BR, DIM), lambda i: (i, 0))] * 3,
        out_shape=[jax.ShapeDtypeStruct((N, DIM), jnp.float32)] * 3,
    )(h_one, W_qkv)


def _logits_body(q_hbm, k_hbm, ei_hbm, ej_hbm, zero8_hbm,
                 expa_hbm, den0_hbm, den1_hbm, pe_hbm, pi_hbm, pj_hbm,
                 cnt_hbm,
                 i_buf, j_buf, east0, east1, qrows0, qrows1, krows0, krows1,
                 trans, iidx0, iidx1, jidx0, jidx1, jadd0, jadd1,
                 pe0, pe1, pi0, pi1, pj0, pj1, cnt_st,
                 den_sh, semq0, semq1, semk0, semk1, semadd0, semadd1,
                 semout0, semout1):
    c = lax.axis_index("c")
    s = lax.axis_index("s")
    w = s * NC + c
    base_e = w * EPW

    east = (east0, east1)
    qrows = (qrows0, qrows1)
    krows = (krows0, krows1)
    iidx = (iidx0, iidx1)
    jidx = (jidx0, jidx1)
    jadd = (jadd0, jadd1)
    semq = (semq0, semq1)
    semk = (semk0, semk1)
    semadd = (semadd0, semadd1)
    semout = (semout0, semout1)

    pltpu.sync_copy(ei_hbm.at[pl.ds(base_e, EPW)], i_buf.at[pl.ds(0, EPW)])
    pltpu.sync_copy(ej_hbm.at[pl.ds(base_e, EPW)], j_buf.at[pl.ds(0, EPW)])
    iota = lax.broadcasted_iota(jnp.int32, (G,), 0)
    for buf in (i_buf, j_buf):
        tv = buf[pl.ds(EPW - 8, G)]
        buf[pl.ds(EPW - 8, G)] = jnp.where(iota < 8, tv, 0)
        for k0 in range(EPW + 8, EBUF, G):
            buf[pl.ds(k0, G)] = jnp.zeros((G,), jnp.int32)
    zstart = jnp.minimum(s * 624, N - 640)
    pltpu.sync_copy(zero8_hbm.at[pl.ds(0, 640)],
                    den_sh.at[pl.ds(zstart, 640)])
    plsc.subcore_barrier()

    def issue(t, p):
        for q in range(CH2 // G):
            off = t * CH2 + q * G
            iidx[p][pl.ds(q * G, G)] = i_buf[pl.ds(off, G)]
            jidx[p][pl.ds(q * G, G)] = j_buf[pl.ds(off, G)]
        for q in range(CH2 // G):
            sl = pl.ds(q * G, G)
            pltpu.async_copy(q_hbm.at[iidx[p].at[sl]], qrows[p].at[sl],
                             semq[p])
            pltpu.async_copy(k_hbm.at[jidx[p].at[sl]], krows[p].at[sl],
                             semk[p])

    def compute(t, p, cnt0, cnt1):
        pltpu.make_async_copy(q_hbm.at[pl.ds(0, CH2)], qrows[p],
                              semq[p]).wait()
        pltpu.make_async_copy(k_hbm.at[pl.ds(0, CH2)], krows[p],
                              semk[p]).wait()
        def group_fn(q, cc):
            cnt0, cnt1 = cc
            off = t * CH2 + q * G
            mask = (off + iota) < EPW
            iv = i_buf[pl.ds(off, G)]
            jv = j_buf[pl.ds(off, G)]
            for e2 in range(G):
                r = q * G + e2
                ce = jnp.full((G,), e2, jnp.int32)
                for h in range(HEADS):
                    qa = qrows[p][r, pl.ds(h * HEAD_DIM, G)]
                    qb = qrows[p][r, pl.ds(h * HEAD_DIM + G, G)]
                    ka = krows[p][r, pl.ds(h * HEAD_DIM, G)]
                    kb = krows[p][r, pl.ds(h * HEAD_DIM + G, G)]
                    ph = qa * ka + qb * kb
                    plsc.store_scatter(trans, [h * G + iota, ce], ph)
            for h in range(HEADS):
                acc = trans[h * G, pl.ds(0, G)]
                for d2 in range(1, G):
                    acc = acc + trans[h * G + d2, pl.ds(0, G)]
                ev = jnp.exp(acc * INV_SQRT_HD)
                ev = jnp.where(mask, ev, 0.0)
                colh = jnp.full((G,), h, jnp.int32)
                plsc.store_scatter(east[p], [q * G + iota, colh], ev)
            epos = w * EBUF + off + iota
            hi = iv >= HALF
            m0 = mask & jnp.logical_not(hi)
            m1 = mask & hi
            inc0 = plsc.cumsum(jnp.where(m0, 1, 0))
            inc1 = plsc.cumsum(jnp.where(m1, 1, 0))
            off0 = cnt0 + inc0 - 1
            off1 = cnt1 + inc1 - 1
            plsc.store_scatter(pe0, [off0], epos, mask=m0)
            plsc.store_scatter(pi0, [off0], iv, mask=m0)
            plsc.store_scatter(pj0, [off0], jv, mask=m0)
            plsc.store_scatter(pe1, [off1], epos, mask=m1)
            plsc.store_scatter(pi1, [off1], iv - HALF, mask=m1)
            plsc.store_scatter(pj1, [off1], jv, mask=m1)
            return (cnt0 + jnp.sum(jnp.where(m0, 1, 0)),
                    cnt1 + jnp.sum(jnp.where(m1, 1, 0)))

        cnt0, cnt1 = lax.fori_loop(0, CH2 // G, group_fn, (cnt0, cnt1))
        for q in range(CH2 // G):
            jadd[p][pl.ds(q * G, G)] = j_buf[pl.ds(t * CH2 + q * G, G)]
        pltpu.async_copy(east[p], den_sh.at[jadd[p]], semadd[p], add=True)
        pltpu.async_copy(east[p],
                         expa_hbm.at[pl.ds(w * EBUF + t * CH2, CH2)],
                         semout[p])
        return cnt0, cnt1

    def drain_add(p):
        pltpu.make_async_copy(zero8_hbm.at[pl.ds(0, CH2)], east[p],
                              semadd[p]).wait()
        pltpu.make_async_copy(zero8_hbm.at[pl.ds(0, CH2)], east[p],
                              semout[p]).wait()

    issue(0, 0)

    def pair_body(t2, carry):
        cnt0, cnt1 = carry
        issue(2 * t2 + 1, 1)

        @pl.when(t2 > 0)
        def _():
            drain_add(0)
        cnt0, cnt1 = compute(2 * t2, 0, cnt0, cnt1)
        issue(2 * t2 + 2, 0)

        @pl.when(t2 > 0)
        def _():
            drain_add(1)
        cnt0, cnt1 = compute(2 * t2 + 1, 1, cnt0, cnt1)
        return cnt0, cnt1

    cnt0, cnt1 = lax.fori_loop(0, NP2, pair_body,
                               (jnp.int32(0), jnp.int32(0)))
    drain_add(0)
    cnt0, cnt1 = compute(NCH - 1, 0, cnt0, cnt1)
    drain_add(0)
    drain_add(1)

    for (cn, pe, pi, pj) in ((cnt0, pe0, pi0, pj0), (cnt1, pe1, pi1, pj1)):
        pcnt = ((cn + 2 * CH2 - 1) // (2 * CH2)) * (2 * CH2)
        zv = jnp.zeros((G,), jnp.int32)
        for k in range(2 * CH2 // G):
            idxp = cn + k * G + iota
            mk = idxp < pcnt
            plsc.store_scatter(pe, [idxp], zv, mask=mk)
            plsc.store_scatter(pi, [idxp], zv, mask=mk)
            plsc.store_scatter(pj, [idxp], zv, mask=mk)

    pltpu.sync_copy(pe0, pe_hbm.at[pl.ds((w * NC + 0) * CAP, CAP)])
    pltpu.sync_copy(pi0, pi_hbm.at[pl.ds((w * NC + 0) * CAP, CAP)])
    pltpu.sync_copy(pj0, pj_hbm.at[pl.ds((w * NC + 0) * CAP, CAP)])
    pltpu.sync_copy(pe1, pe_hbm.at[pl.ds((w * NC + 1) * CAP, CAP)])
    pltpu.sync_copy(pi1, pi_hbm.at[pl.ds((w * NC + 1) * CAP, CAP)])
    pltpu.sync_copy(pj1, pj_hbm.at[pl.ds((w * NC + 1) * CAP, CAP)])
    iota2 = lax.broadcasted_iota(jnp.int32, (G,), 0)
    cnt_st[...] = (jnp.where(iota2 == 0, cnt0, 0)
                   + jnp.where(iota2 == 1, cnt1, 0))
    pltpu.sync_copy(cnt_st, cnt_hbm.at[pl.ds(w * G, G)])

    plsc.subcore_barrier()

    @pl.when(c == 0)
    def _():
        pltpu.sync_copy(den_sh.at[pl.ds(zstart, 640)],
                        den0_hbm.at[pl.ds(zstart, 640)])

    @pl.when(c == 1)
    def _():
        pltpu.sync_copy(den_sh.at[pl.ds(zstart, 640)],
                        den1_hbm.at[pl.ds(zstart, 640)])


def _logits(Q, K, e_e_i, e_e_j):
    zero8 = jnp.zeros((640, HEADS), jnp.float32)
    return pl.kernel(
        _logits_body,
        out_type=(jax.ShapeDtypeStruct((NW * EBUF, HEADS), jnp.float32),
                  jax.ShapeDtypeStruct((N, HEADS), jnp.float32),
                  jax.ShapeDtypeStruct((N, HEADS), jnp.float32),
                  jax.ShapeDtypeStruct((PK_TOTAL,), jnp.int32),
                  jax.ShapeDtypeStruct((PK_TOTAL,), jnp.int32),
                  jax.ShapeDtypeStruct((PK_TOTAL,), jnp.int32),
                  jax.ShapeDtypeStruct((NW * G,), jnp.int32)),
        mesh=_mesh(),
        compiler_params=_sc_params(),
        scratch_types=[
            pltpu.VMEM((EBUF,), jnp.int32),
            pltpu.VMEM((EBUF,), jnp.int32),
            pltpu.VMEM((CH2, HEADS), jnp.float32),
            pltpu.VMEM((CH2, HEADS), jnp.float32),
            pltpu.VMEM((CH2, DIM), jnp.float32),
            pltpu.VMEM((CH2, DIM), jnp.float32),
            pltpu.VMEM((CH2, DIM), jnp.float32),
            pltpu.VMEM((CH2, DIM), jnp.float32),
            pltpu.VMEM((HEADS * G, 17), jnp.float32),
            pltpu.VMEM((CH2,), jnp.int32),
            pltpu.VMEM((CH2,), jnp.int32),
            pltpu.VMEM((CH2,), jnp.int32),
            pltpu.VMEM((CH2,), jnp.int32),
            pltpu.VMEM((CH2,), jnp.int32),
            pltpu.VMEM((CH2,), jnp.int32),
            pltpu.VMEM((CAP,), jnp.int32),
            pltpu.VMEM((CAP,), jnp.int32),
            pltpu.VMEM((CAP,), jnp.int32),
            pltpu.VMEM((CAP,), jnp.int32),
            pltpu.VMEM((CAP,), jnp.int32),
            pltpu.VMEM((CAP,), jnp.int32),
            pltpu.VMEM((G,), jnp.int32),
            pltpu.VMEM_SHARED((N, HEADS), jnp.float32),
            pltpu.SemaphoreType.DMA,
            pltpu.SemaphoreType.DMA,
            pltpu.SemaphoreType.DMA,
            pltpu.SemaphoreType.DMA,
            pltpu.SemaphoreType.DMA,
            pltpu.SemaphoreType.DMA,
            pltpu.SemaphoreType.DMA,
            pltpu.SemaphoreType.DMA,
        ],
    )(Q, K, e_e_i, e_e_j, zero8)


def _attn_body(v_hbm, expa_hbm, den0_hbm, den1_hbm, pe_hbm, pi_hbm, pj_hbm,
               cnt_hbm, zerod_hbm,
               attn_hbm,
               seg_e, seg_i, seg_j,
               vrows0, vrows1, scaled0, scaled1,
               ea0, ea1, d00, d01, d10, d11, w0, w1,
               eidx0, eidx1, jidx0, jidx1, iadd0, iadd1, cnt_st,
               attn_sh,
               semv0, semv1, seme0, seme1, semd00, semd01,
               semd10, semd11, semadd0, semadd1):
    c = lax.axis_index("c")
    s = lax.axis_index("s")

    vrows = (vrows0, vrows1)
    scaled = (scaled0, scaled1)
    ea_st = (ea0, ea1)
    d0_st = (d00, d01)
    d1_st = (d10, d11)
    w_st = (w0, w1)
    eidx = (eidx0, eidx1)
    jidx = (jidx0, jidx1)
    iadd = (iadd0, iadd1)
    semv = (semv0, semv1)
    seme = (seme0, seme1)
    semd0 = (semd00, semd01)
    semd1 = (semd10, semd11)
    semadd = (semadd0, semadd1)

    zs = jnp.minimum(s * 320, HALF - 320)
    pltpu.sync_copy(zerod_hbm.at[pl.ds(0, 320)], attn_sh.at[pl.ds(zs, 320)])
    plsc.subcore_barrier()

    iota = lax.broadcasted_iota(jnp.int32, (G,), 0)

    def drain_add3(p):
        pltpu.make_async_copy(zerod_hbm.at[pl.ds(0, CH)], scaled[p],
                              semadd[p]).wait()

    for li in range(2):
        w = s * NC + li
        lofs = (w * NC + c) * CAP
        pltpu.sync_copy(cnt_hbm.at[pl.ds(w * G, G)], cnt_st)
        cv = cnt_st[...]
        cnt = jnp.where(c == 0, cv[0], cv[1])
        pcnt = ((cnt + 2 * CH - 1) // (2 * CH)) * (2 * CH)
        nseg = (pcnt + SEG - 1) // SEG

        def seg_body(sg, _0, lofs=lofs, cnt=cnt, pcnt=pcnt):
            sbase = sg * SEG
            pltpu.sync_copy(pe_hbm.at[pl.ds(lofs + sbase, SEG)], seg_e)
            pltpu.sync_copy(pi_hbm.at[pl.ds(lofs + sbase, SEG)], seg_i)
            pltpu.sync_copy(pj_hbm.at[pl.ds(lofs + sbase, SEG)], seg_j)
            scnt = jnp.minimum(pcnt - sbase, SEG)
            spairs = scnt // (2 * CH)

            def issue3(t, p):
                for q in range(CH // G):
                    off = t * CH + q * G
                    eidx[p][pl.ds(q * G, G)] = seg_e[pl.ds(off, G)]
                    jidx[p][pl.ds(q * G, G)] = seg_j[pl.ds(off, G)]
                for q in range(CH // G):
                    sl = pl.ds(q * G, G)
                    pltpu.async_copy(v_hbm.at[jidx[p].at[sl]],
                                     vrows[p].at[sl], semv[p])
                    pltpu.async_copy(expa_hbm.at[eidx[p].at[sl]],
                                     ea_st[p].at[sl], seme[p])
                    pltpu.async_copy(den0_hbm.at[jidx[p].at[sl]],
                                     d0_st[p].at[sl], semd0[p])
                    pltpu.async_copy(den1_hbm.at[jidx[p].at[sl]],
                                     d1_st[p].at[sl], semd1[p])

            def compute3(t, p):
                pltpu.make_async_copy(v_hbm.at[pl.ds(0, CH)], vrows[p],
                                      semv[p]).wait()
                pltpu.make_async_copy(expa_hbm.at[pl.ds(0, CH)], ea_st[p],
                                      seme[p]).wait()
                pltpu.make_async_copy(den0_hbm.at[pl.ds(0, CH)], d0_st[p],
                                      semd0[p]).wait()
                pltpu.make_async_copy(den1_hbm.at[pl.ds(0, CH)], d1_st[p],
                                      semd1[p]).wait()
                for q in range(CH // G):
                    off = t * CH + q * G
                    mask = (sbase + off + iota) < cnt
                    row = iota + q * G
                    for h in range(HEADS):
                        colh = jnp.full((G,), h, jnp.int32)
                        eav = plsc.load_gather(ea_st[p], [row, colh])
                        dd = (plsc.load_gather(d0_st[p], [row, colh])
                              + plsc.load_gather(d1_st[p], [row, colh])
                              + 1e-12)
                        wv = jnp.where(mask, eav / dd, 0.0)
                        plsc.store_scatter(w_st[p], [row, colh], wv)
                for e2 in range(CH):
                    wrow = w_st[p][e2, pl.ds(0, G)]
                    for h in range(HEADS):
                        wsc = wrow[h]
                        for r2 in range(2):
                            sl = pl.ds(h * HEAD_DIM + r2 * G, G)
                            scaled[p][e2, sl] = vrows[p][e2, sl] * wsc
                iadd[p][pl.ds(0, G)] = seg_i[pl.ds(t * CH, G)]
                iadd[p][pl.ds(G, G)] = seg_i[pl.ds(t * CH + G, G)]
                pltpu.async_copy(scaled[p], attn_sh.at[iadd[p]],
                                 semadd[p], add=True)

            @pl.when(spairs > 0)
            def _():
                issue3(0, 0)

            def pair3(t2, _2):
                issue3(2 * t2 + 1, 1)

                @pl.when(t2 > 0)
                def _():
                    drain_add3(0)
                compute3(2 * t2, 0)

                @pl.when(2 * t2 + 2 < 2 * spairs)
                def _():
                    issue3(2 * t2 + 2, 0)

                @pl.when(t2 > 0)
                def _():
                    drain_add3(1)
                compute3(2 * t2 + 1, 1)
                return 0

            lax.fori_loop(0, spairs, pair3, 0)

            @pl.when(spairs > 0)
            def _():
                drain_add3(0)
                drain_add3(1)
            return 0

        lax.fori_loop(0, nseg, seg_body, 0)

    plsc.subcore_barrier()
    pltpu.sync_copy(attn_sh.at[pl.ds(zs, 320)],
                    attn_hbm.at[pl.ds(c * HALF + zs, 320)])


def _attn(V, expa, den0, den1, pe, pi, pj, cnts):
    zerod = jnp.zeros((320, DIM), jnp.float32)
    return pl.kernel(
        _attn_body,
        out_type=jax.ShapeDtypeStruct((N, DIM), jnp.float32),
        mesh=_mesh(),
        compiler_params=_sc_params(),
        scratch_types=[
            pltpu.VMEM((SEG,), jnp.int32),
            pltpu.VMEM((SEG,), jnp.int32),
            pltpu.VMEM((SEG,), jnp.int32),
            pltpu.VMEM((CH, DIM), jnp.float32),
            pltpu.VMEM((CH, DIM), jnp.float32),
            pltpu.VMEM((CH, DIM), jnp.float32),
            pltpu.VMEM((CH, DIM), jnp.float32),
            pltpu.VMEM((CH, HEADS), jnp.float32),
            pltpu.VMEM((CH, HEADS), jnp.float32),
            pltpu.VMEM((CH, HEADS), jnp.float32),
            pltpu.VMEM((CH, HEADS), jnp.float32),
            pltpu.VMEM((CH, HEADS), jnp.float32),
            pltpu.VMEM((CH, HEADS), jnp.float32),
            pltpu.VMEM((CH, G + 1), jnp.float32),
            pltpu.VMEM((CH, G + 1), jnp.float32),
            pltpu.VMEM((CH,), jnp.int32),
            pltpu.VMEM((CH,), jnp.int32),
            pltpu.VMEM((CH,), jnp.int32),
            pltpu.VMEM((CH,), jnp.int32),
            pltpu.VMEM((CH,), jnp.int32),
            pltpu.VMEM((CH,), jnp.int32),
            pltpu.VMEM((G,), jnp.int32),
            pltpu.VMEM_SHARED((HALF, DIM), jnp.float32),
            pltpu.SemaphoreType.DMA,
            pltpu.SemaphoreType.DMA,
            pltpu.SemaphoreType.DMA,
            pltpu.SemaphoreType.DMA,
            pltpu.SemaphoreType.DMA,
            pltpu.SemaphoreType.DMA,
            pltpu.SemaphoreType.DMA,
            pltpu.SemaphoreType.DMA,
            pltpu.SemaphoreType.DMA,
            pltpu.SemaphoreType.DMA,
        ],
    )(V, expa, den0, den1, pe, pi, pj, cnts, zerod)


def _layer_norm(x, scale, bias, eps=1e-6):
    mean = jnp.mean(x, axis=-1, keepdims=True)
    var = jnp.mean((x - mean) ** 2, axis=-1, keepdims=True)
    return (x - mean) / jnp.sqrt(var + eps) * scale + bias


def _final_body(h_ref, a_ref, wout_ref, l1s_ref, l1b_ref, wmlp_ref, bmlp_ref,
                l2s_ref, l2b_ref, out_ref):
    h = h_ref[...] + jnp.dot(a_ref[...], wout_ref[...],
                             preferred_element_type=jnp.float32)
    h = _layer_norm(h, l1s_ref[...], l1b_ref[...])
    z = jnp.dot(h, wmlp_ref[...], preferred_element_type=jnp.float32)
    z = z + bmlp_ref[...]
    z = z * (1.0 / (1.0 + jnp.exp(-z)))
    h = h + z
    out_ref[...] = _layer_norm(h, l2s_ref[...], l2b_ref[...])


def _final(h_one, attn, W_out, ln1_s, ln1_b, W_mlp, b_mlp, ln2_s, ln2_b):
    BR = 2000
    vec = pl.BlockSpec((DIM,), lambda i: (0,))
    mat = pl.BlockSpec((DIM, DIM), lambda i: (0, 0))
    row = pl.BlockSpec((BR, DIM), lambda i: (i, 0))
    return pl.pallas_call(
        _final_body,
        grid=(N // BR,),
        in_specs=[row, row, mat, vec, vec, mat, vec, vec, vec],
        out_specs=row,
        out_shape=jax.ShapeDtypeStruct((N, DIM), jnp.float32),
    )(h_one, attn, W_out, ln1_s, ln1_b, W_mlp, b_mlp, ln2_s, ln2_b)


def kernel(h_one, e_e_i, e_e_j, W_qkv, W_out, ln1_scale, ln1_bias,
           W_mlp, b_mlp, ln2_scale, ln2_bias):
    Q, K, V = _qkv(h_one, W_qkv)
    expa, den0, den1, pe, pi, pj, cnts = _logits(Q, K, e_e_i, e_e_j)
    attn = _attn(V, expa, den0, den1, pe, pi, pj, cnts)
    return _final(h_one, attn, W_out, ln1_scale, ln1_bias,
                  W_mlp, b_mlp, ln2_scale, ln2_bias)


# speedup vs baseline: 1.0597x; 1.0597x over previous
import functools
import math

import jax
import jax.numpy as jnp
from jax import lax
from jax.experimental import pallas as pl
from jax.experimental.pallas import tpu as pltpu
from jax.experimental.pallas import tpu_sc as plsc

N = 10000
DIM = 256
HEADS = 8
HEAD_DIM = DIM // HEADS
E = 160000

NC = 2
NS = 16
NW = NC * NS
EPW = E // NW
G = 16
CH2 = 64
NCH = (EPW + CH2 - 1) // CH2
NP2 = (NCH - 1) // 2
EBUF = NCH * CH2
CH = 32
HALF = N // NC
CAP = ((EPW + 2 * CH2 - 1) // (2 * CH2)) * (2 * CH2)
SEG = 2048
PK_TOTAL = NW * NC * CAP + SEG
INV_SQRT_HD = 1.0 / math.sqrt(HEAD_DIM)

_mesh = functools.partial(
    plsc.VectorSubcoreMesh,
    core_axis_name="c", subcore_axis_name="s", num_cores=NC, num_subcores=NS)

_sc_params = functools.partial(
    pltpu.CompilerParams, use_tc_tiling_on_sc=False, needs_layout_passes=False)


def _qkv_body(h_ref, w_ref, q_ref, k_ref, v_ref):
    y = jnp.dot(h_ref[...], w_ref[...], preferred_element_type=jnp.float32)
    q_ref[...] = y[:, :DIM].astype(jnp.bfloat16)
    k_ref[...] = y[:, DIM:2 * DIM].astype(jnp.bfloat16)
    v_ref[...] = y[:, 2 * DIM:]


def _qkv(h_one, W_qkv):
    BR = 2000
    return pl.pallas_call(
        _qkv_body,
        grid=(N // BR,),
        in_specs=[pl.BlockSpec((BR, DIM), lambda i: (i, 0)),
                  pl.BlockSpec((DIM, 3 * DIM), lambda i: (0, 0))],
        out_specs=[pl.BlockSpec((BR, DIM), lambda i: (i, 0))] * 3,
        out_shape=[jax.ShapeDtypeStruct((N, DIM), jnp.bfloat16),
                   jax.ShapeDtypeStruct((N, DIM), jnp.bfloat16),
                   jax.ShapeDtypeStruct((N, DIM), jnp.float32)],
    )(h_one, W_qkv)


def _logits_body(q_hbm, k_hbm, ei_hbm, ej_hbm, zero8_hbm,
                 expa_hbm, den0_hbm, den1_hbm, pe_hbm, pi_hbm, pj_hbm,
                 cnt_hbm,
                 i_buf, j_buf, east0, east1, qrows0, qrows1, krows0, krows1,
                 trans, iidx0, iidx1, jidx0, jidx1, jadd0, jadd1,
                 pe0, pe1, pi0, pi1, pj0, pj1, cnt_st,
                 den_sh, semq0, semq1, semk0, semk1, semadd0, semadd1,
                 semout0, semout1):
    c = lax.axis_index("c")
    s = lax.axis_index("s")
    w = s * NC + c
    base_e = w * EPW

    east = (east0, east1)
    qrows = (qrows0, qrows1)
    krows = (krows0, krows1)
    iidx = (iidx0, iidx1)
    jidx = (jidx0, jidx1)
    jadd = (jadd0, jadd1)
    semq = (semq0, semq1)
    semk = (semk0, semk1)
    semadd = (semadd0, semadd1)
    semout = (semout0, semout1)

    pltpu.sync_copy(ei_hbm.at[pl.ds(base_e, EPW)], i_buf.at[pl.ds(0, EPW)])
    pltpu.sync_copy(ej_hbm.at[pl.ds(base_e, EPW)], j_buf.at[pl.ds(0, EPW)])
    iota = lax.broadcasted_iota(jnp.int32, (G,), 0)
    for buf in (i_buf, j_buf):
        tv = buf[pl.ds(EPW - 8, G)]
        buf[pl.ds(EPW - 8, G)] = jnp.where(iota < 8, tv, 0)
        for k0 in range(EPW + 8, EBUF, G):
            buf[pl.ds(k0, G)] = jnp.zeros((G,), jnp.int32)
    zstart = jnp.minimum(s * 624, N - 640)
    pltpu.sync_copy(zero8_hbm.at[pl.ds(0, 640)],
                    den_sh.at[pl.ds(zstart, 640)])
    plsc.subcore_barrier()

    def issue(t, p):
        for q in range(CH2 // G):
            off = t * CH2 + q * G
            iidx[p][pl.ds(q * G, G)] = i_buf[pl.ds(off, G)]
            jidx[p][pl.ds(q * G, G)] = j_buf[pl.ds(off, G)]
        pltpu.async_copy(q_hbm.at[iidx[p]], qrows[p], semq[p])
        pltpu.async_copy(k_hbm.at[jidx[p]], krows[p], semk[p])

    def compute(t, p, cnt0, cnt1):
        pltpu.make_async_copy(q_hbm.at[pl.ds(0, CH2)], qrows[p],
                              semq[p]).wait()
        pltpu.make_async_copy(k_hbm.at[pl.ds(0, CH2)], krows[p],
                              semk[p]).wait()
        def group_fn(q, cc):
            cnt0, cnt1 = cc
            off = t * CH2 + q * G
            mask = (off + iota) < EPW
            iv = i_buf[pl.ds(off, G)]
            jv = j_buf[pl.ds(off, G)]
            for e2 in range(G):
                r = q * G + e2
                ce = jnp.full((G,), e2, jnp.int32)
                for h in range(HEADS):
                    qh = qrows[p][r, pl.ds(h * HEAD_DIM, HEAD_DIM)]
                    kh = krows[p][r, pl.ds(h * HEAD_DIM, HEAD_DIM)]
                    qa, qb = plsc.unpack(qh, format=plsc.PackFormat.INTERLEAVED)
                    ka, kb = plsc.unpack(kh, format=plsc.PackFormat.INTERLEAVED)
                    ph = qa * ka + qb * kb
                    plsc.store_scatter(trans, [h * G + iota, ce], ph)
            for h in range(HEADS):
                acc = trans[h * G, pl.ds(0, G)]
                for d2 in range(1, G):
                    acc = acc + trans[h * G + d2, pl.ds(0, G)]
                ev = jnp.exp(acc * INV_SQRT_HD)
                ev = jnp.where(mask, ev, 0.0)
                colh = jnp.full((G,), h, jnp.int32)
                plsc.store_scatter(east[p], [q * G + iota, colh], ev)
            epos = w * EBUF + off + iota
            hi = iv >= HALF
            m0 = mask & jnp.logical_not(hi)
            m1 = mask & hi
            inc0 = plsc.cumsum(jnp.where(m0, 1, 0))
            inc1 = plsc.cumsum(jnp.where(m1, 1, 0))
            off0 = cnt0 + inc0 - 1
            off1 = cnt1 + inc1 - 1
            plsc.store_scatter(pe0, [off0], epos, mask=m0)
            plsc.store_scatter(pi0, [off0], iv, mask=m0)
            plsc.store_scatter(pj0, [off0], jv, mask=m0)
            plsc.store_scatter(pe1, [off1], epos, mask=m1)
            plsc.store_scatter(pi1, [off1], iv - HALF, mask=m1)
            plsc.store_scatter(pj1, [off1], jv, mask=m1)
            return (cnt0 + jnp.sum(jnp.where(m0, 1, 0)),
                    cnt1 + jnp.sum(jnp.where(m1, 1, 0)))

        cnt0, cnt1 = lax.fori_loop(0, CH2 // G, group_fn, (cnt0, cnt1))
        for q in range(CH2 // G):
            jadd[p][pl.ds(q * G, G)] = j_buf[pl.ds(t * CH2 + q * G, G)]
        pltpu.async_copy(east[p], den_sh.at[jadd[p]], semadd[p], add=True)
        pltpu.async_copy(east[p],
                         expa_hbm.at[pl.ds(w * EBUF + t * CH2, CH2)],
                         semout[p])
        return cnt0, cnt1

    def drain_add(p):
        pltpu.make_async_copy(zero8_hbm.at[pl.ds(0, CH2)], east[p],
                              semadd[p]).wait()
        pltpu.make_async_copy(zero8_hbm.at[pl.ds(0, CH2)], east[p],
                              semout[p]).wait()

    issue(0, 0)

    def pair_body(t2, carry):
        cnt0, cnt1 = carry
        issue(2 * t2 + 1, 1)

        @pl.when(t2 > 0)
        def _():
            drain_add(0)
        cnt0, cnt1 = compute(2 * t2, 0, cnt0, cnt1)
        issue(2 * t2 + 2, 0)

        @pl.when(t2 > 0)
        def _():
            drain_add(1)
        cnt0, cnt1 = compute(2 * t2 + 1, 1, cnt0, cnt1)
        return cnt0, cnt1

    cnt0, cnt1 = lax.fori_loop(0, NP2, pair_body,
                               (jnp.int32(0), jnp.int32(0)))
    drain_add(0)
    cnt0, cnt1 = compute(NCH - 1, 0, cnt0, cnt1)
    drain_add(0)
    drain_add(1)

    for (cn, pe, pi, pj) in ((cnt0, pe0, pi0, pj0), (cnt1, pe1, pi1, pj1)):
        pcnt = ((cn + 2 * CH2 - 1) // (2 * CH2)) * (2 * CH2)
        zv = jnp.zeros((G,), jnp.int32)
        for k in range(2 * CH2 // G):
            idxp = cn + k * G + iota
            mk = idxp < pcnt
            plsc.store_scatter(pe, [idxp], zv, mask=mk)
            plsc.store_scatter(pi, [idxp], zv, mask=mk)
            plsc.store_scatter(pj, [idxp], zv, mask=mk)

    pltpu.sync_copy(pe0, pe_hbm.at[pl.ds((w * NC + 0) * CAP, CAP)])
    pltpu.sync_copy(pi0, pi_hbm.at[pl.ds((w * NC + 0) * CAP, CAP)])
    pltpu.sync_copy(pj0, pj_hbm.at[pl.ds((w * NC + 0) * CAP, CAP)])
    pltpu.sync_copy(pe1, pe_hbm.at[pl.ds((w * NC + 1) * CAP, CAP)])
    pltpu.sync_copy(pi1, pi_hbm.at[pl.ds((w * NC + 1) * CAP, CAP)])
    pltpu.sync_copy(pj1, pj_hbm.at[pl.ds((w * NC + 1) * CAP, CAP)])
    iota2 = lax.broadcasted_iota(jnp.int32, (G,), 0)
    cnt_st[...] = (jnp.where(iota2 == 0, cnt0, 0)
                   + jnp.where(iota2 == 1, cnt1, 0))
    pltpu.sync_copy(cnt_st, cnt_hbm.at[pl.ds(w * G, G)])

    plsc.subcore_barrier()

    @pl.when(c == 0)
    def _():
        pltpu.sync_copy(den_sh.at[pl.ds(zstart, 640)],
                        den0_hbm.at[pl.ds(zstart, 640)])

    @pl.when(c == 1)
    def _():
        pltpu.sync_copy(den_sh.at[pl.ds(zstart, 640)],
                        den1_hbm.at[pl.ds(zstart, 640)])


def _logits(Q, K, e_e_i, e_e_j):
    zero8 = jnp.zeros((640, HEADS), jnp.float32)
    return pl.kernel(
        _logits_body,
        out_type=(jax.ShapeDtypeStruct((NW * EBUF, HEADS), jnp.float32),
                  jax.ShapeDtypeStruct((N, HEADS), jnp.float32),
                  jax.ShapeDtypeStruct((N, HEADS), jnp.float32),
                  jax.ShapeDtypeStruct((PK_TOTAL,), jnp.int32),
                  jax.ShapeDtypeStruct((PK_TOTAL,), jnp.int32),
                  jax.ShapeDtypeStruct((PK_TOTAL,), jnp.int32),
                  jax.ShapeDtypeStruct((NW * G,), jnp.int32)),
        mesh=_mesh(),
        compiler_params=_sc_params(),
        scratch_types=[
            pltpu.VMEM((EBUF,), jnp.int32),
            pltpu.VMEM((EBUF,), jnp.int32),
            pltpu.VMEM((CH2, HEADS), jnp.float32),
            pltpu.VMEM((CH2, HEADS), jnp.float32),
            pltpu.VMEM((CH2, DIM), jnp.bfloat16),
            pltpu.VMEM((CH2, DIM), jnp.bfloat16),
            pltpu.VMEM((CH2, DIM), jnp.bfloat16),
            pltpu.VMEM((CH2, DIM), jnp.bfloat16),
            pltpu.VMEM((HEADS * G, 17), jnp.float32),
            pltpu.VMEM((CH2,), jnp.int32),
            pltpu.VMEM((CH2,), jnp.int32),
            pltpu.VMEM((CH2,), jnp.int32),
            pltpu.VMEM((CH2,), jnp.int32),
            pltpu.VMEM((CH2,), jnp.int32),
            pltpu.VMEM((CH2,), jnp.int32),
            pltpu.VMEM((CAP,), jnp.int32),
            pltpu.VMEM((CAP,), jnp.int32),
            pltpu.VMEM((CAP,), jnp.int32),
            pltpu.VMEM((CAP,), jnp.int32),
            pltpu.VMEM((CAP,), jnp.int32),
            pltpu.VMEM((CAP,), jnp.int32),
            pltpu.VMEM((G,), jnp.int32),
            pltpu.VMEM_SHARED((N, HEADS), jnp.float32),
            pltpu.SemaphoreType.DMA,
            pltpu.SemaphoreType.DMA,
            pltpu.SemaphoreType.DMA,
            pltpu.SemaphoreType.DMA,
            pltpu.SemaphoreType.DMA,
            pltpu.SemaphoreType.DMA,
            pltpu.SemaphoreType.DMA,
            pltpu.SemaphoreType.DMA,
        ],
    )(Q, K, e_e_i, e_e_j, zero8)


def _attn_body(v_hbm, expa_hbm, den0_hbm, den1_hbm, pe_hbm, pi_hbm, pj_hbm,
               cnt_hbm, zerod_hbm,
               attn_hbm,
               seg_e, seg_i, seg_j,
               vrows0, vrows1, scaled0, scaled1,
               ea0, ea1, d00, d01, d10, d11, w0, w1,
               eidx0, eidx1, jidx0, jidx1, iadd0, iadd1, cnt_st,
               attn_sh,
               semv0, semv1, seme0, seme1, semd00, semd01,
               semd10, semd11, semadd0, semadd1):
    c = lax.axis_index("c")
    s = lax.axis_index("s")

    vrows = (vrows0, vrows1)
    scaled = (scaled0, scaled1)
    ea_st = (ea0, ea1)
    d0_st = (d00, d01)
    d1_st = (d10, d11)
    w_st = (w0, w1)
    eidx = (eidx0, eidx1)
    jidx = (jidx0, jidx1)
    iadd = (iadd0, iadd1)
    semv = (semv0, semv1)
    seme = (seme0, seme1)
    semd0 = (semd00, semd01)
    semd1 = (semd10, semd11)
    semadd = (semadd0, semadd1)

    zs = jnp.minimum(s * 320, HALF - 320)
    pltpu.sync_copy(zerod_hbm.at[pl.ds(0, 320)], attn_sh.at[pl.ds(zs, 320)])
    plsc.subcore_barrier()

    iota = lax.broadcasted_iota(jnp.int32, (G,), 0)

    def drain_add3(p):
        pltpu.make_async_copy(zerod_hbm.at[pl.ds(0, CH)], scaled[p],
                              semadd[p]).wait()

    for li in range(2):
        w = s * NC + li
        lofs = (w * NC + c) * CAP
        pltpu.sync_copy(cnt_hbm.at[pl.ds(w * G, G)], cnt_st)
        cv = cnt_st[...]
        cnt = jnp.where(c == 0, cv[0], cv[1])
        pcnt = ((cnt + 2 * CH - 1) // (2 * CH)) * (2 * CH)
        nseg = (pcnt + SEG - 1) // SEG

        def seg_body(sg, _0, lofs=lofs, cnt=cnt, pcnt=pcnt):
            sbase = sg * SEG
            pltpu.sync_copy(pe_hbm.at[pl.ds(lofs + sbase, SEG)], seg_e)
            pltpu.sync_copy(pi_hbm.at[pl.ds(lofs + sbase, SEG)], seg_i)
            pltpu.sync_copy(pj_hbm.at[pl.ds(lofs + sbase, SEG)], seg_j)
            scnt = jnp.minimum(pcnt - sbase, SEG)
            spairs = scnt // (2 * CH)

            def issue3(t, p):
                for q in range(CH // G):
                    off = t * CH + q * G
                    eidx[p][pl.ds(q * G, G)] = seg_e[pl.ds(off, G)]
                    jidx[p][pl.ds(q * G, G)] = seg_j[pl.ds(off, G)]
                pltpu.async_copy(v_hbm.at[jidx[p]], vrows[p], semv[p])
                pltpu.async_copy(expa_hbm.at[eidx[p]], ea_st[p], seme[p])
                pltpu.async_copy(den0_hbm.at[jidx[p]], d0_st[p], semd0[p])
                pltpu.async_copy(den1_hbm.at[jidx[p]], d1_st[p], semd1[p])

            def compute3(t, p):
                pltpu.make_async_copy(v_hbm.at[pl.ds(0, CH)], vrows[p],
                                      semv[p]).wait()
                pltpu.make_async_copy(expa_hbm.at[pl.ds(0, CH)], ea_st[p],
                                      seme[p]).wait()
                pltpu.make_async_copy(den0_hbm.at[pl.ds(0, CH)], d0_st[p],
                                      semd0[p]).wait()
                pltpu.make_async_copy(den1_hbm.at[pl.ds(0, CH)], d1_st[p],
                                      semd1[p]).wait()
                for q in range(CH // G):
                    off = t * CH + q * G
                    mask = (sbase + off + iota) < cnt
                    row = iota + q * G
                    for h in range(HEADS):
                        colh = jnp.full((G,), h, jnp.int32)
                        eav = plsc.load_gather(ea_st[p], [row, colh])
                        dd = (plsc.load_gather(d0_st[p], [row, colh])
                              + plsc.load_gather(d1_st[p], [row, colh])
                              + 1e-12)
                        wv = jnp.where(mask, eav / dd, 0.0)
                        plsc.store_scatter(w_st[p], [row, colh], wv)
                for e2 in range(CH):
                    wrow = w_st[p][e2, pl.ds(0, G)]
                    for h in range(HEADS):
                        wsc = wrow[h]
                        for r2 in range(2):
                            sl = pl.ds(h * HEAD_DIM + r2 * G, G)
                            scaled[p][e2, sl] = vrows[p][e2, sl] * wsc
                iadd[p][pl.ds(0, G)] = seg_i[pl.ds(t * CH, G)]
                iadd[p][pl.ds(G, G)] = seg_i[pl.ds(t * CH + G, G)]
                pltpu.async_copy(scaled[p], attn_sh.at[iadd[p]],
                                 semadd[p], add=True)

            @pl.when(spairs > 0)
            def _():
                issue3(0, 0)

            def pair3(t2, _2):
                issue3(2 * t2 + 1, 1)

                @pl.when(t2 > 0)
                def _():
                    drain_add3(0)
                compute3(2 * t2, 0)

                @pl.when(2 * t2 + 2 < 2 * spairs)
                def _():
                    issue3(2 * t2 + 2, 0)

                @pl.when(t2 > 0)
                def _():
                    drain_add3(1)
                compute3(2 * t2 + 1, 1)
                return 0

            lax.fori_loop(0, spairs, pair3, 0)

            @pl.when(spairs > 0)
            def _():
                drain_add3(0)
                drain_add3(1)
            return 0

        lax.fori_loop(0, nseg, seg_body, 0)

    plsc.subcore_barrier()
    pltpu.sync_copy(attn_sh.at[pl.ds(zs, 320)],
                    attn_hbm.at[pl.ds(c * HALF + zs, 320)])


def _attn(V, expa, den0, den1, pe, pi, pj, cnts):
    zerod = jnp.zeros((320, DIM), jnp.float32)
    return pl.kernel(
        _attn_body,
        out_type=jax.ShapeDtypeStruct((N, DIM), jnp.float32),
        mesh=_mesh(),
        compiler_params=_sc_params(),
        scratch_types=[
            pltpu.VMEM((SEG,), jnp.int32),
            pltpu.VMEM((SEG,), jnp.int32),
            pltpu.VMEM((SEG,), jnp.int32),
            pltpu.VMEM((CH, DIM), jnp.float32),
            pltpu.VMEM((CH, DIM), jnp.float32),
            pltpu.VMEM((CH, DIM), jnp.float32),
            pltpu.VMEM((CH, DIM), jnp.float32),
            pltpu.VMEM((CH, HEADS), jnp.float32),
            pltpu.VMEM((CH, HEADS), jnp.float32),
            pltpu.VMEM((CH, HEADS), jnp.float32),
            pltpu.VMEM((CH, HEADS), jnp.float32),
            pltpu.VMEM((CH, HEADS), jnp.float32),
            pltpu.VMEM((CH, HEADS), jnp.float32),
            pltpu.VMEM((CH, G + 1), jnp.float32),
            pltpu.VMEM((CH, G + 1), jnp.float32),
            pltpu.VMEM((CH,), jnp.int32),
            pltpu.VMEM((CH,), jnp.int32),
            pltpu.VMEM((CH,), jnp.int32),
            pltpu.VMEM((CH,), jnp.int32),
            pltpu.VMEM((CH,), jnp.int32),
            pltpu.VMEM((CH,), jnp.int32),
            pltpu.VMEM((G,), jnp.int32),
            pltpu.VMEM_SHARED((HALF, DIM), jnp.float32),
            pltpu.SemaphoreType.DMA,
            pltpu.SemaphoreType.DMA,
            pltpu.SemaphoreType.DMA,
            pltpu.SemaphoreType.DMA,
            pltpu.SemaphoreType.DMA,
            pltpu.SemaphoreType.DMA,
            pltpu.SemaphoreType.DMA,
            pltpu.SemaphoreType.DMA,
            pltpu.SemaphoreType.DMA,
            pltpu.SemaphoreType.DMA,
        ],
    )(V, expa, den0, den1, pe, pi, pj, cnts, zerod)


def _layer_norm(x, scale, bias, eps=1e-6):
    mean = jnp.mean(x, axis=-1, keepdims=True)
    var = jnp.mean((x - mean) ** 2, axis=-1, keepdims=True)
    return (x - mean) / jnp.sqrt(var + eps) * scale + bias


def _final_body(h_ref, a_ref, wout_ref, l1s_ref, l1b_ref, wmlp_ref, bmlp_ref,
                l2s_ref, l2b_ref, out_ref):
    h = h_ref[...] + jnp.dot(a_ref[...], wout_ref[...],
                             preferred_element_type=jnp.float32)
    h = _layer_norm(h, l1s_ref[...], l1b_ref[...])
    z = jnp.dot(h, wmlp_ref[...], preferred_element_type=jnp.float32)
    z = z + bmlp_ref[...]
    z = z * (1.0 / (1.0 + jnp.exp(-z)))
    h = h + z
    out_ref[...] = _layer_norm(h, l2s_ref[...], l2b_ref[...])


def _final(h_one, attn, W_out, ln1_s, ln1_b, W_mlp, b_mlp, ln2_s, ln2_b):
    BR = 2000
    vec = pl.BlockSpec((DIM,), lambda i: (0,))
    mat = pl.BlockSpec((DIM, DIM), lambda i: (0, 0))
    row = pl.BlockSpec((BR, DIM), lambda i: (i, 0))
    return pl.pallas_call(
        _final_body,
        grid=(N // BR,),
        in_specs=[row, row, mat, vec, vec, mat, vec, vec, vec],
        out_specs=row,
        out_shape=jax.ShapeDtypeStruct((N, DIM), jnp.float32),
    )(h_one, attn, W_out, ln1_s, ln1_b, W_mlp, b_mlp, ln2_s, ln2_b)


def kernel(h_one, e_e_i, e_e_j, W_qkv, W_out, ln1_scale, ln1_bias,
           W_mlp, b_mlp, ln2_scale, ln2_bias):
    Q, K, V = _qkv(h_one, W_qkv)
    expa, den0, den1, pe, pi, pj, cnts = _logits(Q, K, e_e_i, e_e_j)
    attn = _attn(V, expa, den0, den1, pe, pi, pj, cnts)
    return _final(h_one, attn, W_out, ln1_scale, ln1_bias,
                  W_mlp, b_mlp, ln2_scale, ln2_bias)


# speedup vs baseline: 1.1757x; 1.1094x over previous
import functools
import math

import jax
import jax.numpy as jnp
from jax import lax
from jax.experimental import pallas as pl
from jax.experimental.pallas import tpu as pltpu
from jax.experimental.pallas import tpu_sc as plsc

N = 10000
DIM = 256
HEADS = 8
HEAD_DIM = DIM // HEADS
E = 160000

NC = 2
NS = 16
NW = NC * NS
EPW = E // NW
G = 16
CH2 = 64
NCH = (EPW + CH2 - 1) // CH2
NP2 = (NCH - 1) // 2
EBUF = NCH * CH2
CH = 32
HALF = N // NC
CAP = ((EPW + 2 * CH2 - 1) // (2 * CH2)) * (2 * CH2)
SEG = 2048
PK_TOTAL = NW * NC * CAP + SEG
INV_SQRT_HD = 1.0 / math.sqrt(HEAD_DIM)

_mesh = functools.partial(
    plsc.VectorSubcoreMesh,
    core_axis_name="c", subcore_axis_name="s", num_cores=NC, num_subcores=NS)

_sc_params = functools.partial(
    pltpu.CompilerParams, use_tc_tiling_on_sc=False, needs_layout_passes=False)


def _qkv_body(h_ref, w_ref, q_ref, k_ref, v_ref):
    y = jnp.dot(h_ref[...], w_ref[...], preferred_element_type=jnp.float32)
    q_ref[...] = y[:, :DIM].astype(jnp.bfloat16)
    k_ref[...] = y[:, DIM:2 * DIM].astype(jnp.bfloat16)
    v_ref[...] = y[:, 2 * DIM:]


def _qkv(h_one, W_qkv):
    BR = 2000
    return pl.pallas_call(
        _qkv_body,
        grid=(N // BR,),
        in_specs=[pl.BlockSpec((BR, DIM), lambda i: (i, 0)),
                  pl.BlockSpec((DIM, 3 * DIM), lambda i: (0, 0))],
        out_specs=[pl.BlockSpec((BR, DIM), lambda i: (i, 0))] * 3,
        out_shape=[jax.ShapeDtypeStruct((N, DIM), jnp.bfloat16),
                   jax.ShapeDtypeStruct((N, DIM), jnp.bfloat16),
                   jax.ShapeDtypeStruct((N, DIM), jnp.float32)],
    )(h_one, W_qkv)


def _logits_body(q_hbm, k_hbm, ei_hbm, ej_hbm, zero8_hbm,
                 expa_hbm, den0_hbm, den1_hbm, pe_hbm, pi_hbm, pj_hbm,
                 cnt_hbm,
                 i_buf, j_buf, east0, east1, qrows0, qrows1, krows0, krows1,
                 trans, iidx0, iidx1, jidx0, jidx1, jadd0, jadd1,
                 pe0, pe1, pi0, pi1, pj0, pj1, cnt_st,
                 den_sh, semq0, semq1, semk0, semk1, semadd0, semadd1,
                 semout0, semout1):
    c = lax.axis_index("c")
    s = lax.axis_index("s")
    w = s * NC + c
    base_e = w * EPW

    east = (east0, east1)
    qrows = (qrows0, qrows1)
    krows = (krows0, krows1)
    iidx = (iidx0, iidx1)
    jidx = (jidx0, jidx1)
    jadd = (jadd0, jadd1)
    semq = (semq0, semq1)
    semk = (semk0, semk1)
    semadd = (semadd0, semadd1)
    semout = (semout0, semout1)

    pltpu.sync_copy(ei_hbm.at[pl.ds(base_e, EPW)], i_buf.at[pl.ds(0, EPW)])
    pltpu.sync_copy(ej_hbm.at[pl.ds(base_e, EPW)], j_buf.at[pl.ds(0, EPW)])
    iota = lax.broadcasted_iota(jnp.int32, (G,), 0)
    for buf in (i_buf, j_buf):
        tv = buf[pl.ds(EPW - 8, G)]
        buf[pl.ds(EPW - 8, G)] = jnp.where(iota < 8, tv, 0)
        for k0 in range(EPW + 8, EBUF, G):
            buf[pl.ds(k0, G)] = jnp.zeros((G,), jnp.int32)
    zstart = jnp.minimum(s * 624, N - 640)
    pltpu.sync_copy(zero8_hbm.at[pl.ds(0, 640)],
                    den_sh.at[pl.ds(zstart, 640)])
    plsc.subcore_barrier()

    def issue(t, p):
        for q in range(CH2 // G):
            off = t * CH2 + q * G
            iidx[p][pl.ds(q * G, G)] = i_buf[pl.ds(off, G)]
            jidx[p][pl.ds(q * G, G)] = j_buf[pl.ds(off, G)]
        pltpu.async_copy(q_hbm.at[iidx[p]], qrows[p], semq[p])
        pltpu.async_copy(k_hbm.at[jidx[p]], krows[p], semk[p])

    def compute(t, p, cnt0, cnt1):
        pltpu.make_async_copy(q_hbm.at[pl.ds(0, CH2)], qrows[p],
                              semq[p]).wait()
        pltpu.make_async_copy(k_hbm.at[pl.ds(0, CH2)], krows[p],
                              semk[p]).wait()
        def group_fn(q, cc):
            cnt0, cnt1 = cc
            off = t * CH2 + q * G
            mask = (off + iota) < EPW
            iv = i_buf[pl.ds(off, G)]
            jv = j_buf[pl.ds(off, G)]
            idx17 = [iota * 17 + rr for rr in range(8)]
            for h in range(HEADS):
                for e2 in range(G):
                    r = q * G + e2
                    qh = qrows[p][r, pl.ds(h * HEAD_DIM, HEAD_DIM)]
                    kh = krows[p][r, pl.ds(h * HEAD_DIM, HEAD_DIM)]
                    qa, qb = plsc.unpack(qh, format=plsc.PackFormat.INTERLEAVED)
                    ka, kb = plsc.unpack(kh, format=plsc.PackFormat.INTERLEAVED)
                    ph = qa * ka + qb * kb
                    plsc.store_scatter(
                        trans.at[pl.ds(h * G * 17 + (e2 // 8) * 8, 264)],
                        [idx17[e2 % 8]], ph)
            for h in range(HEADS):
                acc = trans[pl.ds((h * G) * 17, G)]
                for d2 in range(1, G):
                    acc = acc + trans[pl.ds((h * G + d2) * 17, G)]
                ev = jnp.exp(acc * INV_SQRT_HD)
                ev = jnp.where(mask, ev, 0.0)
                colh = jnp.full((G,), h, jnp.int32)
                plsc.store_scatter(east[p], [q * G + iota, colh], ev)
            epos = w * EBUF + off + iota
            hi = iv >= HALF
            m0 = mask & jnp.logical_not(hi)
            m1 = mask & hi
            inc0 = plsc.cumsum(jnp.where(m0, 1, 0))
            inc1 = plsc.cumsum(jnp.where(m1, 1, 0))
            off0 = cnt0 + inc0 - 1
            off1 = cnt1 + inc1 - 1
            plsc.store_scatter(pe0, [off0], epos, mask=m0)
            plsc.store_scatter(pi0, [off0], iv, mask=m0)
            plsc.store_scatter(pj0, [off0], jv, mask=m0)
            plsc.store_scatter(pe1, [off1], epos, mask=m1)
            plsc.store_scatter(pi1, [off1], iv - HALF, mask=m1)
            plsc.store_scatter(pj1, [off1], jv, mask=m1)
            return (cnt0 + jnp.sum(jnp.where(m0, 1, 0)),
                    cnt1 + jnp.sum(jnp.where(m1, 1, 0)))

        cnt0, cnt1 = lax.fori_loop(0, CH2 // G, group_fn, (cnt0, cnt1))
        for q in range(CH2 // G):
            jadd[p][pl.ds(q * G, G)] = j_buf[pl.ds(t * CH2 + q * G, G)]
        pltpu.async_copy(east[p], den_sh.at[jadd[p]], semadd[p], add=True)
        pltpu.async_copy(east[p],
                         expa_hbm.at[pl.ds(w * EBUF + t * CH2, CH2)],
                         semout[p])
        return cnt0, cnt1

    def drain_add(p):
        pltpu.make_async_copy(zero8_hbm.at[pl.ds(0, CH2)], east[p],
                              semadd[p]).wait()
        pltpu.make_async_copy(zero8_hbm.at[pl.ds(0, CH2)], east[p],
                              semout[p]).wait()

    issue(0, 0)

    def pair_body(t2, carry):
        cnt0, cnt1 = carry
        issue(2 * t2 + 1, 1)

        @pl.when(t2 > 0)
        def _():
            drain_add(0)
        cnt0, cnt1 = compute(2 * t2, 0, cnt0, cnt1)
        issue(2 * t2 + 2, 0)

        @pl.when(t2 > 0)
        def _():
            drain_add(1)
        cnt0, cnt1 = compute(2 * t2 + 1, 1, cnt0, cnt1)
        return cnt0, cnt1

    cnt0, cnt1 = lax.fori_loop(0, NP2, pair_body,
                               (jnp.int32(0), jnp.int32(0)))
    drain_add(0)
    cnt0, cnt1 = compute(NCH - 1, 0, cnt0, cnt1)
    drain_add(0)
    drain_add(1)

    for (cn, pe, pi, pj) in ((cnt0, pe0, pi0, pj0), (cnt1, pe1, pi1, pj1)):
        pcnt = ((cn + 2 * CH2 - 1) // (2 * CH2)) * (2 * CH2)
        zv = jnp.zeros((G,), jnp.int32)
        for k in range(2 * CH2 // G):
            idxp = cn + k * G + iota
            mk = idxp < pcnt
            plsc.store_scatter(pe, [idxp], zv, mask=mk)
            plsc.store_scatter(pi, [idxp], zv, mask=mk)
            plsc.store_scatter(pj, [idxp], zv, mask=mk)

    pltpu.sync_copy(pe0, pe_hbm.at[pl.ds((w * NC + 0) * CAP, CAP)])
    pltpu.sync_copy(pi0, pi_hbm.at[pl.ds((w * NC + 0) * CAP, CAP)])
    pltpu.sync_copy(pj0, pj_hbm.at[pl.ds((w * NC + 0) * CAP, CAP)])
    pltpu.sync_copy(pe1, pe_hbm.at[pl.ds((w * NC + 1) * CAP, CAP)])
    pltpu.sync_copy(pi1, pi_hbm.at[pl.ds((w * NC + 1) * CAP, CAP)])
    pltpu.sync_copy(pj1, pj_hbm.at[pl.ds((w * NC + 1) * CAP, CAP)])
    iota2 = lax.broadcasted_iota(jnp.int32, (G,), 0)
    cnt_st[...] = (jnp.where(iota2 == 0, cnt0, 0)
                   + jnp.where(iota2 == 1, cnt1, 0))
    pltpu.sync_copy(cnt_st, cnt_hbm.at[pl.ds(w * G, G)])

    plsc.subcore_barrier()

    @pl.when(c == 0)
    def _():
        pltpu.sync_copy(den_sh.at[pl.ds(zstart, 640)],
                        den0_hbm.at[pl.ds(zstart, 640)])

    @pl.when(c == 1)
    def _():
        pltpu.sync_copy(den_sh.at[pl.ds(zstart, 640)],
                        den1_hbm.at[pl.ds(zstart, 640)])


def _logits(Q, K, e_e_i, e_e_j):
    zero8 = jnp.zeros((640, HEADS), jnp.float32)
    return pl.kernel(
        _logits_body,
        out_type=(jax.ShapeDtypeStruct((NW * EBUF, HEADS), jnp.float32),
                  jax.ShapeDtypeStruct((N, HEADS), jnp.float32),
                  jax.ShapeDtypeStruct((N, HEADS), jnp.float32),
                  jax.ShapeDtypeStruct((PK_TOTAL,), jnp.int32),
                  jax.ShapeDtypeStruct((PK_TOTAL,), jnp.int32),
                  jax.ShapeDtypeStruct((PK_TOTAL,), jnp.int32),
                  jax.ShapeDtypeStruct((NW * G,), jnp.int32)),
        mesh=_mesh(),
        compiler_params=_sc_params(),
        scratch_types=[
            pltpu.VMEM((EBUF,), jnp.int32),
            pltpu.VMEM((EBUF,), jnp.int32),
            pltpu.VMEM((CH2, HEADS), jnp.float32),
            pltpu.VMEM((CH2, HEADS), jnp.float32),
            pltpu.VMEM((CH2, DIM), jnp.bfloat16),
            pltpu.VMEM((CH2, DIM), jnp.bfloat16),
            pltpu.VMEM((CH2, DIM), jnp.bfloat16),
            pltpu.VMEM((CH2, DIM), jnp.bfloat16),
            pltpu.VMEM((HEADS * G * 17,), jnp.float32),
            pltpu.VMEM((CH2,), jnp.int32),
            pltpu.VMEM((CH2,), jnp.int32),
            pltpu.VMEM((CH2,), jnp.int32),
            pltpu.VMEM((CH2,), jnp.int32),
            pltpu.VMEM((CH2,), jnp.int32),
            pltpu.VMEM((CH2,), jnp.int32),
            pltpu.VMEM((CAP,), jnp.int32),
            pltpu.VMEM((CAP,), jnp.int32),
            pltpu.VMEM((CAP,), jnp.int32),
            pltpu.VMEM((CAP,), jnp.int32),
            pltpu.VMEM((CAP,), jnp.int32),
            pltpu.VMEM((CAP,), jnp.int32),
            pltpu.VMEM((G,), jnp.int32),
            pltpu.VMEM_SHARED((N, HEADS), jnp.float32),
            pltpu.SemaphoreType.DMA,
            pltpu.SemaphoreType.DMA,
            pltpu.SemaphoreType.DMA,
            pltpu.SemaphoreType.DMA,
            pltpu.SemaphoreType.DMA,
            pltpu.SemaphoreType.DMA,
            pltpu.SemaphoreType.DMA,
            pltpu.SemaphoreType.DMA,
        ],
    )(Q, K, e_e_i, e_e_j, zero8)


def _attn_body(v_hbm, expa_hbm, den0_hbm, den1_hbm, pe_hbm, pi_hbm, pj_hbm,
               cnt_hbm, zerod_hbm,
               attn_hbm,
               seg_e, seg_i, seg_j,
               vrows0, vrows1, scaled0, scaled1,
               ea0, ea1, d00, d01, d10, d11, w0, w1,
               eidx0, eidx1, jidx0, jidx1, iadd0, iadd1, cnt_st,
               attn_sh,
               semv0, semv1, seme0, seme1, semd00, semd01,
               semd10, semd11, semadd0, semadd1):
    c = lax.axis_index("c")
    s = lax.axis_index("s")

    vrows = (vrows0, vrows1)
    scaled = (scaled0, scaled1)
    ea_st = (ea0, ea1)
    d0_st = (d00, d01)
    d1_st = (d10, d11)
    w_st = (w0, w1)
    eidx = (eidx0, eidx1)
    jidx = (jidx0, jidx1)
    iadd = (iadd0, iadd1)
    semv = (semv0, semv1)
    seme = (seme0, seme1)
    semd0 = (semd00, semd01)
    semd1 = (semd10, semd11)
    semadd = (semadd0, semadd1)

    zs = jnp.minimum(s * 320, HALF - 320)
    pltpu.sync_copy(zerod_hbm.at[pl.ds(0, 320)], attn_sh.at[pl.ds(zs, 320)])
    plsc.subcore_barrier()

    iota = lax.broadcasted_iota(jnp.int32, (G,), 0)

    def drain_add3(p):
        pltpu.make_async_copy(zerod_hbm.at[pl.ds(0, CH)], scaled[p],
                              semadd[p]).wait()

    for li in range(2):
        w = s * NC + li
        lofs = (w * NC + c) * CAP
        pltpu.sync_copy(cnt_hbm.at[pl.ds(w * G, G)], cnt_st)
        cv = cnt_st[...]
        cnt = jnp.where(c == 0, cv[0], cv[1])
        pcnt = ((cnt + 2 * CH - 1) // (2 * CH)) * (2 * CH)
        nseg = (pcnt + SEG - 1) // SEG

        def seg_body(sg, _0, lofs=lofs, cnt=cnt, pcnt=pcnt):
            sbase = sg * SEG
            pltpu.sync_copy(pe_hbm.at[pl.ds(lofs + sbase, SEG)], seg_e)
            pltpu.sync_copy(pi_hbm.at[pl.ds(lofs + sbase, SEG)], seg_i)
            pltpu.sync_copy(pj_hbm.at[pl.ds(lofs + sbase, SEG)], seg_j)
            scnt = jnp.minimum(pcnt - sbase, SEG)
            spairs = scnt // (2 * CH)

            def issue3(t, p):
                for q in range(CH // G):
                    off = t * CH + q * G
                    eidx[p][pl.ds(q * G, G)] = seg_e[pl.ds(off, G)]
                    jidx[p][pl.ds(q * G, G)] = seg_j[pl.ds(off, G)]
                pltpu.async_copy(v_hbm.at[jidx[p]], vrows[p], semv[p])
                pltpu.async_copy(expa_hbm.at[eidx[p]], ea_st[p], seme[p])
                pltpu.async_copy(den0_hbm.at[jidx[p]], d0_st[p], semd0[p])
                pltpu.async_copy(den1_hbm.at[jidx[p]], d1_st[p], semd1[p])

            def compute3(t, p):
                pltpu.make_async_copy(v_hbm.at[pl.ds(0, CH)], vrows[p],
                                      semv[p]).wait()
                pltpu.make_async_copy(expa_hbm.at[pl.ds(0, CH)], ea_st[p],
                                      seme[p]).wait()
                pltpu.make_async_copy(den0_hbm.at[pl.ds(0, CH)], d0_st[p],
                                      semd0[p]).wait()
                pltpu.make_async_copy(den1_hbm.at[pl.ds(0, CH)], d1_st[p],
                                      semd1[p]).wait()
                for q in range(CH // G):
                    off = t * CH + q * G
                    mask = (sbase + off + iota) < cnt
                    row = iota + q * G
                    for h in range(HEADS):
                        colh = jnp.full((G,), h, jnp.int32)
                        eav = plsc.load_gather(ea_st[p], [row, colh])
                        dd = (plsc.load_gather(d0_st[p], [row, colh])
                              + plsc.load_gather(d1_st[p], [row, colh])
                              + 1e-12)
                        wv = jnp.where(mask, eav / dd, 0.0)
                        plsc.store_scatter(w_st[p], [row, colh], wv)
                for e2 in range(CH):
                    wrow = w_st[p][e2, pl.ds(0, G)]
                    for h in range(HEADS):
                        wsc = wrow[h]
                        for r2 in range(2):
                            sl = pl.ds(h * HEAD_DIM + r2 * G, G)
                            scaled[p][e2, sl] = vrows[p][e2, sl] * wsc
                iadd[p][pl.ds(0, G)] = seg_i[pl.ds(t * CH, G)]
                iadd[p][pl.ds(G, G)] = seg_i[pl.ds(t * CH + G, G)]
                pltpu.async_copy(scaled[p], attn_sh.at[iadd[p]],
                                 semadd[p], add=True)

            @pl.when(spairs > 0)
            def _():
                issue3(0, 0)

            def pair3(t2, _2):
                issue3(2 * t2 + 1, 1)

                @pl.when(t2 > 0)
                def _():
                    drain_add3(0)
                compute3(2 * t2, 0)

                @pl.when(2 * t2 + 2 < 2 * spairs)
                def _():
                    issue3(2 * t2 + 2, 0)

                @pl.when(t2 > 0)
                def _():
                    drain_add3(1)
                compute3(2 * t2 + 1, 1)
                return 0

            lax.fori_loop(0, spairs, pair3, 0)

            @pl.when(spairs > 0)
            def _():
                drain_add3(0)
                drain_add3(1)
            return 0

        lax.fori_loop(0, nseg, seg_body, 0)

    plsc.subcore_barrier()
    pltpu.sync_copy(attn_sh.at[pl.ds(zs, 320)],
                    attn_hbm.at[pl.ds(c * HALF + zs, 320)])


def _attn(V, expa, den0, den1, pe, pi, pj, cnts):
    zerod = jnp.zeros((320, DIM), jnp.float32)
    return pl.kernel(
        _attn_body,
        out_type=jax.ShapeDtypeStruct((N, DIM), jnp.float32),
        mesh=_mesh(),
        compiler_params=_sc_params(),
        scratch_types=[
            pltpu.VMEM((SEG,), jnp.int32),
            pltpu.VMEM((SEG,), jnp.int32),
            pltpu.VMEM((SEG,), jnp.int32),
            pltpu.VMEM((CH, DIM), jnp.float32),
            pltpu.VMEM((CH, DIM), jnp.float32),
            pltpu.VMEM((CH, DIM), jnp.float32),
            pltpu.VMEM((CH, DIM), jnp.float32),
            pltpu.VMEM((CH, HEADS), jnp.float32),
            pltpu.VMEM((CH, HEADS), jnp.float32),
            pltpu.VMEM((CH, HEADS), jnp.float32),
            pltpu.VMEM((CH, HEADS), jnp.float32),
            pltpu.VMEM((CH, HEADS), jnp.float32),
            pltpu.VMEM((CH, HEADS), jnp.float32),
            pltpu.VMEM((CH, G + 1), jnp.float32),
            pltpu.VMEM((CH, G + 1), jnp.float32),
            pltpu.VMEM((CH,), jnp.int32),
            pltpu.VMEM((CH,), jnp.int32),
            pltpu.VMEM((CH,), jnp.int32),
            pltpu.VMEM((CH,), jnp.int32),
            pltpu.VMEM((CH,), jnp.int32),
            pltpu.VMEM((CH,), jnp.int32),
            pltpu.VMEM((G,), jnp.int32),
            pltpu.VMEM_SHARED((HALF, DIM), jnp.float32),
            pltpu.SemaphoreType.DMA,
            pltpu.SemaphoreType.DMA,
            pltpu.SemaphoreType.DMA,
            pltpu.SemaphoreType.DMA,
            pltpu.SemaphoreType.DMA,
            pltpu.SemaphoreType.DMA,
            pltpu.SemaphoreType.DMA,
            pltpu.SemaphoreType.DMA,
            pltpu.SemaphoreType.DMA,
            pltpu.SemaphoreType.DMA,
        ],
    )(V, expa, den0, den1, pe, pi, pj, cnts, zerod)


def _layer_norm(x, scale, bias, eps=1e-6):
    mean = jnp.mean(x, axis=-1, keepdims=True)
    var = jnp.mean((x - mean) ** 2, axis=-1, keepdims=True)
    return (x - mean) / jnp.sqrt(var + eps) * scale + bias


def _final_body(h_ref, a_ref, wout_ref, l1s_ref, l1b_ref, wmlp_ref, bmlp_ref,
                l2s_ref, l2b_ref, out_ref):
    h = h_ref[...] + jnp.dot(a_ref[...], wout_ref[...],
                             preferred_element_type=jnp.float32)
    h = _layer_norm(h, l1s_ref[...], l1b_ref[...])
    z = jnp.dot(h, wmlp_ref[...], preferred_element_type=jnp.float32)
    z = z + bmlp_ref[...]
    z = z * (1.0 / (1.0 + jnp.exp(-z)))
    h = h + z
    out_ref[...] = _layer_norm(h, l2s_ref[...], l2b_ref[...])


def _final(h_one, attn, W_out, ln1_s, ln1_b, W_mlp, b_mlp, ln2_s, ln2_b):
    BR = 2000
    vec = pl.BlockSpec((DIM,), lambda i: (0,))
    mat = pl.BlockSpec((DIM, DIM), lambda i: (0, 0))
    row = pl.BlockSpec((BR, DIM), lambda i: (i, 0))
    return pl.pallas_call(
        _final_body,
        grid=(N // BR,),
        in_specs=[row, row, mat, vec, vec, mat, vec, vec, vec],
        out_specs=row,
        out_shape=jax.ShapeDtypeStruct((N, DIM), jnp.float32),
    )(h_one, attn, W_out, ln1_s, ln1_b, W_mlp, b_mlp, ln2_s, ln2_b)


def kernel(h_one, e_e_i, e_e_j, W_qkv, W_out, ln1_scale, ln1_bias,
           W_mlp, b_mlp, ln2_scale, ln2_bias):
    Q, K, V = _qkv(h_one, W_qkv)
    expa, den0, den1, pe, pi, pj, cnts = _logits(Q, K, e_e_i, e_e_j)
    attn = _attn(V, expa, den0, den1, pe, pi, pj, cnts)
    return _final(h_one, attn, W_out, ln1_scale, ln1_bias,
                  W_mlp, b_mlp, ln2_scale, ln2_bias)
